# baseline (speedup 1.0000x reference)
import functools

import jax
import jax.numpy as jnp
from jax import lax
from jax.experimental import pallas as pl
from jax.experimental.pallas import tpu as pltpu
from jax.experimental.pallas import tpu_sc as plsc

DIM = 4096
N_EXPERTS = 64
TOPK = 8
BT = 1024
NW = 32
LANES = 16
NCHUNK = 1
UNROLL = 4


def _score_kernel(x_ref, w_ref, p_ref):
    x = x_ref[...]
    w = w_ref[...]
    scores = jax.lax.dot_general(
        w, x, (((1,), (1,)), ((), ())), preferred_element_type=jnp.float32
    )
    m = jnp.max(scores, axis=0, keepdims=True)
    e = jnp.exp(scores - m)
    probs = e / jnp.sum(e, axis=0, keepdims=True)
    nsub, ch = p_ref.shape[0], p_ref.shape[2]
    for k in range(nsub):
        p_ref[k] = probs[:, k * ch:(k + 1) * ch]


def _tc_probs(x, weight, ch):
    n_tokens = x.shape[0]
    return pl.pallas_call(
        _score_kernel,
        grid=(n_tokens // BT,),
        in_specs=[
            pl.BlockSpec((BT, DIM), lambda i: (i, 0)),
            pl.BlockSpec((N_EXPERTS, DIM), lambda i: (0, 0)),
        ],
        out_specs=pl.BlockSpec((BT // ch, N_EXPERTS, ch), lambda i: (i, 0, 0)),
        out_shape=jax.ShapeDtypeStruct(
            (n_tokens // ch, N_EXPERTS, ch), jnp.float32
        ),
    )(x, weight)


@functools.cache
def _sc_topk_build(ch):
    mesh = plsc.VectorSubcoreMesh(core_axis_name="c", subcore_axis_name="s")

    @functools.partial(
        pl.kernel,
        mesh=mesh,
        out_type=[
            jax.ShapeDtypeStruct((NW, TOPK, ch), jnp.float32),
            jax.ShapeDtypeStruct((NW, TOPK, ch), jnp.int32),
        ],
        scratch_types=[
            pltpu.VMEM((N_EXPERTS, ch), jnp.float32),
            pltpu.VMEM((TOPK, ch), jnp.float32),
            pltpu.VMEM((TOPK, ch), jnp.int32),
        ],
    )
    def sc_topk(p_hbm, wout_hbm, iout_hbm, p_v, w_v, i_v):
        wid = lax.axis_index("s") * 2 + lax.axis_index("c")
        pltpu.sync_copy(p_hbm.at[wid], p_v)

        NG = ch // LANES
        NI = 2

        def group_body(g, _):
            sls = [
                pl.ds(pl.multiple_of((g * NI + k) * LANES, LANES), LANES)
                for k in range(NI)
            ]
            init = (
                tuple(jnp.full((LANES,), -1.0, jnp.float32)
                      for _ in range(NI * TOPK)),
                tuple(jnp.zeros((LANES,), jnp.int32)
                      for _ in range(NI * TOPK)),
            )

            def expert_body(i, carry):
                vals, idxs = carry
                for u in range(UNROLL):
                    e = N_EXPERTS - 1 - (i * UNROLL + u)
                    ei = jnp.full((LANES,), e, jnp.int32)
                    nv_l, ni_l = [], []
                    for k in range(NI):
                        v = p_v[e, sls[k]]
                        eik = ei
                        for j in range(TOPK):
                            jj = k * TOPK + j
                            swap = v >= vals[jj]
                            nv = jnp.where(swap, v, vals[jj])
                            pv = jnp.where(swap, vals[jj], v)
                            ni = jnp.where(swap, eik, idxs[jj])
                            pi = jnp.where(swap, idxs[jj], eik)
                            nv_l.append(nv)
                            ni_l.append(ni)
                            v, eik = pv, pi
                    vals, idxs = tuple(nv_l), tuple(ni_l)
                return vals, idxs

            vals, idxs = lax.fori_loop(
                0, N_EXPERTS // UNROLL, expert_body, init
            )
            for k in range(NI):
                for j in range(TOPK):
                    w_v[j, sls[k]] = vals[k * TOPK + j]
                    i_v[j, sls[k]] = idxs[k * TOPK + j]
            return 0

        lax.fori_loop(0, NG // NI, group_body, 0)
        pltpu.sync_copy(w_v, wout_hbm.at[wid])
        pltpu.sync_copy(i_v, iout_hbm.at[wid])

    return sc_topk


def kernel(x, weight):
    n_tokens = x.shape[0]
    nchunk = n_tokens // NCHUNK
    ch = nchunk // NW
    sc_topk = _sc_topk_build(ch)
    probs = [
        _tc_probs(x[c * nchunk:(c + 1) * nchunk], weight, ch)
        for c in range(NCHUNK)
    ]
    outs = [sc_topk(p) for p in probs]
    ws = [w.transpose(0, 2, 1).reshape(nchunk, TOPK) for w, _ in outs]
    is_ = [i.transpose(0, 2, 1).reshape(nchunk, TOPK) for _, i in outs]
    return jnp.concatenate(ws, 0), jnp.concatenate(is_, 0)


# speedup vs baseline: 1.0442x; 1.0442x over previous
import functools

import jax
import jax.numpy as jnp
from jax import lax
from jax.experimental import pallas as pl
from jax.experimental.pallas import tpu as pltpu
from jax.experimental.pallas import tpu_sc as plsc

DIM = 4096
N_EXPERTS = 64
TOPK = 8
BT = 1024
NW = 32
LANES = 16
UNROLL = 4


def _score_topk_kernel(x_ref, w_ref, p_ref, wt_ref, it_ref):
    x = x_ref[...]
    w = w_ref[...]
    scores = jax.lax.dot_general(
        w, x, (((1,), (1,)), ((), ())), preferred_element_type=jnp.float32
    )
    m = jnp.max(scores, axis=0, keepdims=True)
    e = jnp.exp(scores - m)
    probs = e / jnp.sum(e, axis=0, keepdims=True)

    nsub, ch = p_ref.shape[0], p_ref.shape[2]
    for k in range(nsub):
        p_ref[k] = probs[:, k * ch:(k + 1) * ch]

    iota = jax.lax.broadcasted_iota(jnp.int32, probs.shape, 0)
    s = probs
    vals, idxs = [], []
    for k in range(TOPK):
        mx = jnp.max(s, axis=0, keepdims=True)
        idx = jnp.min(jnp.where(s == mx, iota, N_EXPERTS), axis=0, keepdims=True)
        vals.append(mx)
        idxs.append(idx)
        if k + 1 < TOPK:
            s = jnp.where(iota == idx, -1.0, s)
    wt_ref[...] = jnp.concatenate(vals, axis=0)
    it_ref[...] = jnp.concatenate(idxs, axis=0)


def _tc_stage(x, weight, ch, nb_half):
    n_tokens = x.shape[0]
    nb = n_tokens // BT
    return pl.pallas_call(
        _score_topk_kernel,
        grid=(nb,),
        in_specs=[
            pl.BlockSpec((BT, DIM), lambda i: (i, 0)),
            pl.BlockSpec((N_EXPERTS, DIM), lambda i: (0, 0)),
        ],
        out_specs=[
            pl.BlockSpec((BT // ch, N_EXPERTS, ch), lambda i: (i % nb_half, 0, 0)),
            pl.BlockSpec((TOPK, BT), lambda i: (0, jnp.minimum(i, nb_half))),
            pl.BlockSpec((TOPK, BT), lambda i: (0, jnp.minimum(i, nb_half))),
        ],
        out_shape=[
            jax.ShapeDtypeStruct((nb_half * BT // ch, N_EXPERTS, ch), jnp.float32),
            jax.ShapeDtypeStruct((TOPK, (nb_half + 1) * BT), jnp.float32),
            jax.ShapeDtypeStruct((TOPK, (nb_half + 1) * BT), jnp.int32),
        ],
    )(x, weight)


@functools.cache
def _sc_topk_build(ch):
    mesh = plsc.VectorSubcoreMesh(core_axis_name="c", subcore_axis_name="s")

    @functools.partial(
        pl.kernel,
        mesh=mesh,
        out_type=[
            jax.ShapeDtypeStruct((NW, TOPK, ch), jnp.float32),
            jax.ShapeDtypeStruct((NW, TOPK, ch), jnp.int32),
        ],
        scratch_types=[
            pltpu.VMEM((N_EXPERTS, ch), jnp.float32),
            pltpu.VMEM((TOPK, ch), jnp.float32),
            pltpu.VMEM((TOPK, ch), jnp.int32),
        ],
    )
    def sc_topk(p_hbm, wout_hbm, iout_hbm, p_v, w_v, i_v):
        wid = lax.axis_index("s") * 2 + lax.axis_index("c")
        pltpu.sync_copy(p_hbm.at[wid], p_v)

        NG = ch // LANES
        NI = 2

        def group_body(g, _):
            sls = [
                pl.ds(pl.multiple_of((g * NI + k) * LANES, LANES), LANES)
                for k in range(NI)
            ]
            init = (
                tuple(jnp.full((LANES,), -1.0, jnp.float32)
                      for _ in range(NI * TOPK)),
                tuple(jnp.zeros((LANES,), jnp.int32)
                      for _ in range(NI * TOPK)),
            )

            def expert_body(i, carry):
                vals, idxs = carry
                for u in range(UNROLL):
                    e = N_EXPERTS - 1 - (i * UNROLL + u)
                    ei = jnp.full((LANES,), e, jnp.int32)
                    nv_l, ni_l = [], []
                    for k in range(NI):
                        v = p_v[e, sls[k]]
                        eik = ei
                        for j in range(TOPK):
                            jj = k * TOPK + j
                            swap = v >= vals[jj]
                            nv = jnp.where(swap, v, vals[jj])
                            pv = jnp.where(swap, vals[jj], v)
                            ni = jnp.where(swap, eik, idxs[jj])
                            pi = jnp.where(swap, idxs[jj], eik)
                            nv_l.append(nv)
                            ni_l.append(ni)
                            v, eik = pv, pi
                    vals, idxs = tuple(nv_l), tuple(ni_l)
                return vals, idxs

            vals, idxs = lax.fori_loop(
                0, N_EXPERTS // UNROLL, expert_body, init
            )
            for k in range(NI):
                for j in range(TOPK):
                    w_v[j, sls[k]] = vals[k * TOPK + j]
                    i_v[j, sls[k]] = idxs[k * TOPK + j]
            return 0

        lax.fori_loop(0, NG // NI, group_body, 0)
        pltpu.sync_copy(w_v, wout_hbm.at[wid])
        pltpu.sync_copy(i_v, iout_hbm.at[wid])

    return sc_topk


def kernel(x, weight):
    n_tokens = x.shape[0]
    half = n_tokens // 2
    nb_half = half // BT
    ch = half // NW
    probs2, wt, it = _tc_stage(x, weight, ch, nb_half)
    scw, sci = _sc_topk_build(ch)(probs2)
    w1 = wt[:, :half].T
    i1 = it[:, :half].T
    w2 = scw.transpose(0, 2, 1).reshape(half, TOPK)
    i2 = sci.transpose(0, 2, 1).reshape(half, TOPK)
    return jnp.concatenate([w1, w2], 0), jnp.concatenate([i1, i2], 0)


# speedup vs baseline: 1.0924x; 1.0462x over previous
import functools

import jax
import jax.numpy as jnp
from jax import lax
from jax.experimental import pallas as pl
from jax.experimental.pallas import tpu as pltpu
from jax.experimental.pallas import tpu_sc as plsc

DIM = 4096
N_EXPERTS = 64
TOPK = 8
BT = 1024
NW = 32
LANES = 16
UNROLL = 4


def _score_topk_kernel(x_ref, w_ref, p_ref, wt_ref, it_ref):
    x = x_ref[...]
    w = w_ref[...]
    scores = jax.lax.dot_general(
        w, x, (((1,), (1,)), ((), ())), preferred_element_type=jnp.float32
    )
    m = jnp.max(scores, axis=0, keepdims=True)
    e = jnp.exp(scores - m)
    probs = e / jnp.sum(e, axis=0, keepdims=True)

    nsub, ch = p_ref.shape[0], p_ref.shape[2]
    for k in range(nsub):
        p_ref[k] = probs[:, k * ch:(k + 1) * ch]

    iota = jax.lax.broadcasted_iota(jnp.int32, probs.shape, 0)
    s = probs
    vals, idxs = [], []
    for k in range(TOPK):
        mx = jnp.max(s, axis=0, keepdims=True)
        idx = jnp.min(jnp.where(s == mx, iota, N_EXPERTS), axis=0, keepdims=True)
        vals.append(mx)
        idxs.append(idx)
        if k + 1 < TOPK:
            s = jnp.where(iota == idx, -1.0, s)
    wt_ref[...] = jnp.concatenate(vals, axis=0)
    it_ref[...] = jnp.concatenate(idxs, axis=0)


def _tc_stage(x, weight, ch, nb_tc, nb_sc):
    n_tokens = x.shape[0]
    nb = n_tokens // BT
    return pl.pallas_call(
        _score_topk_kernel,
        grid=(nb,),
        in_specs=[
            pl.BlockSpec((BT, DIM), lambda i: (i, 0)),
            pl.BlockSpec((N_EXPERTS, DIM), lambda i: (0, 0)),
        ],
        out_specs=[
            pl.BlockSpec((BT // ch, N_EXPERTS, ch), lambda i: (i % nb_sc, 0, 0)),
            pl.BlockSpec((TOPK, BT), lambda i: (0, jnp.minimum(i, nb_tc))),
            pl.BlockSpec((TOPK, BT), lambda i: (0, jnp.minimum(i, nb_tc))),
        ],
        out_shape=[
            jax.ShapeDtypeStruct((nb_sc * BT // ch, N_EXPERTS, ch), jnp.float32),
            jax.ShapeDtypeStruct((TOPK, (nb_tc + 1) * BT), jnp.float32),
            jax.ShapeDtypeStruct((TOPK, (nb_tc + 1) * BT), jnp.int32),
        ],
    )(x, weight)


@functools.cache
def _sc_topk_build(ch):
    mesh = plsc.VectorSubcoreMesh(core_axis_name="c", subcore_axis_name="s")

    @functools.partial(
        pl.kernel,
        mesh=mesh,
        out_type=[
            jax.ShapeDtypeStruct((NW, TOPK, ch), jnp.float32),
            jax.ShapeDtypeStruct((NW, TOPK, ch), jnp.int32),
        ],
        scratch_types=[
            pltpu.VMEM((N_EXPERTS, ch), jnp.float32),
            pltpu.VMEM((TOPK, ch), jnp.float32),
            pltpu.VMEM((TOPK, ch), jnp.int32),
        ],
    )
    def sc_topk(p_hbm, wout_hbm, iout_hbm, p_v, w_v, i_v):
        wid = lax.axis_index("s") * 2 + lax.axis_index("c")
        pltpu.sync_copy(p_hbm.at[wid], p_v)

        NG = ch // LANES
        NI = 2

        def group_body(g, _):
            sls = [
                pl.ds(pl.multiple_of((g * NI + k) * LANES, LANES), LANES)
                for k in range(NI)
            ]
            init = (
                tuple(jnp.full((LANES,), -1.0, jnp.float32)
                      for _ in range(NI * TOPK)),
                tuple(jnp.zeros((LANES,), jnp.int32)
                      for _ in range(NI * TOPK)),
            )

            def expert_body(i, carry):
                vals, idxs = carry
                for u in range(UNROLL):
                    e = N_EXPERTS - 1 - (i * UNROLL + u)
                    ei = jnp.full((LANES,), e, jnp.int32)
                    nv_l, ni_l = [], []
                    for k in range(NI):
                        v = p_v[e, sls[k]]
                        eik = ei
                        for j in range(TOPK):
                            jj = k * TOPK + j
                            swap = v >= vals[jj]
                            nv = jnp.where(swap, v, vals[jj])
                            pv = jnp.where(swap, vals[jj], v)
                            ni = jnp.where(swap, eik, idxs[jj])
                            pi = jnp.where(swap, idxs[jj], eik)
                            nv_l.append(nv)
                            ni_l.append(ni)
                            v, eik = pv, pi
                    vals, idxs = tuple(nv_l), tuple(ni_l)
                return vals, idxs

            vals, idxs = lax.fori_loop(
                0, N_EXPERTS // UNROLL, expert_body, init
            )
            for k in range(NI):
                for j in range(TOPK):
                    w_v[j, sls[k]] = vals[k * TOPK + j]
                    i_v[j, sls[k]] = idxs[k * TOPK + j]
            return 0

        lax.fori_loop(0, NG // NI, group_body, 0)
        pltpu.sync_copy(w_v, wout_hbm.at[wid])
        pltpu.sync_copy(i_v, iout_hbm.at[wid])

    return sc_topk


SC_BLOCKS = 4


def kernel(x, weight):
    n_tokens = x.shape[0]
    sc_tok = SC_BLOCKS * BT
    tc_tok = n_tokens - sc_tok
    nb_tc = tc_tok // BT
    ch = sc_tok // NW
    probs2, wt, it = _tc_stage(x, weight, ch, nb_tc, SC_BLOCKS)
    scw, sci = _sc_topk_build(ch)(probs2)
    w1 = wt[:, :tc_tok].T
    i1 = it[:, :tc_tok].T
    w2 = scw.transpose(0, 2, 1).reshape(sc_tok, TOPK)
    i2 = sci.transpose(0, 2, 1).reshape(sc_tok, TOPK)
    return jnp.concatenate([w1, w2], 0), jnp.concatenate([i1, i2], 0)


# speedup vs baseline: 1.0968x; 1.0040x over previous
import functools

import jax
import jax.numpy as jnp
from jax import lax
from jax.experimental import pallas as pl
from jax.experimental.pallas import tpu as pltpu
from jax.experimental.pallas import tpu_sc as plsc

DIM = 4096
N_EXPERTS = 64
TOPK = 8
BT = 1024
NW = 32
LANES = 16
UNROLL = 4


def _score_topk_kernel(x_ref, w_ref, p_ref, wt_ref, it_ref):
    x = x_ref[...]
    w = w_ref[...]
    scores = jax.lax.dot_general(
        w, x, (((1,), (1,)), ((), ())), preferred_element_type=jnp.float32
    )
    m = jnp.max(scores, axis=0, keepdims=True)
    e = jnp.exp(scores - m)
    probs = e / jnp.sum(e, axis=0, keepdims=True)

    nsub, ch = p_ref.shape[0], p_ref.shape[2]
    for k in range(nsub):
        p_ref[k] = probs[:, k * ch:(k + 1) * ch]

    iota = jax.lax.broadcasted_iota(jnp.int32, probs.shape, 0)
    s = probs
    vals, idxs = [], []
    for k in range(TOPK):
        mx = jnp.max(s, axis=0, keepdims=True)
        idx = jnp.min(jnp.where(s == mx, iota, N_EXPERTS), axis=0, keepdims=True)
        vals.append(mx)
        idxs.append(idx)
        if k + 1 < TOPK:
            s = jnp.where(iota == idx, -1.0, s)
    wt_ref[...] = jnp.concatenate(vals, axis=0)
    it_ref[...] = jnp.concatenate(idxs, axis=0)


def _tc_stage(x, weight, ch, nb_tc, nb_sc):
    n_tokens = x.shape[0]
    nb = n_tokens // BT
    return pl.pallas_call(
        _score_topk_kernel,
        grid=(nb,),
        in_specs=[
            pl.BlockSpec((BT, DIM), lambda i: (i, 0)),
            pl.BlockSpec((N_EXPERTS, DIM), lambda i: (0, 0)),
        ],
        out_specs=[
            pl.BlockSpec((BT // ch, N_EXPERTS, ch), lambda i: (i % nb_sc, 0, 0)),
            pl.BlockSpec((TOPK, BT), lambda i: (0, jnp.minimum(i, nb_tc))),
            pl.BlockSpec((TOPK, BT), lambda i: (0, jnp.minimum(i, nb_tc))),
        ],
        out_shape=[
            jax.ShapeDtypeStruct((nb_sc * BT // ch, N_EXPERTS, ch), jnp.float32),
            jax.ShapeDtypeStruct((TOPK, (nb_tc + 1) * BT), jnp.float32),
            jax.ShapeDtypeStruct((TOPK, (nb_tc + 1) * BT), jnp.int32),
        ],
    )(x, weight)


@functools.cache
def _sc_topk_build(ch):
    mesh = plsc.VectorSubcoreMesh(core_axis_name="c", subcore_axis_name="s")

    @functools.partial(
        pl.kernel,
        mesh=mesh,
        out_type=[
            jax.ShapeDtypeStruct((NW, TOPK, ch), jnp.float32),
            jax.ShapeDtypeStruct((NW, TOPK, ch), jnp.int32),
        ],
        scratch_types=[
            pltpu.VMEM((N_EXPERTS, ch), jnp.float32),
            pltpu.VMEM((TOPK, ch), jnp.float32),
            pltpu.VMEM((TOPK, ch), jnp.int32),
        ],
    )
    def sc_topk(p_hbm, wout_hbm, iout_hbm, p_v, w_v, i_v):
        wid = lax.axis_index("s") * 2 + lax.axis_index("c")
        pltpu.sync_copy(p_hbm.at[wid], p_v)

        NG = ch // LANES
        NI = 2

        def group_body(g, _):
            sls = [
                pl.ds(pl.multiple_of((g * NI + k) * LANES, LANES), LANES)
                for k in range(NI)
            ]
            init = (
                tuple(jnp.full((LANES,), -1.0, jnp.float32)
                      for _ in range(NI * TOPK)),
                tuple(jnp.zeros((LANES,), jnp.int32)
                      for _ in range(NI * TOPK)),
            )

            def expert_body(i, carry):
                vals, idxs = carry
                for u in range(UNROLL):
                    e = N_EXPERTS - 1 - (i * UNROLL + u)
                    ei = jnp.full((LANES,), e, jnp.int32)
                    nv_l, ni_l = [], []
                    for k in range(NI):
                        v = p_v[e, sls[k]]
                        eik = ei
                        for j in range(TOPK):
                            jj = k * TOPK + j
                            swap = v >= vals[jj]
                            nv = jnp.where(swap, v, vals[jj])
                            pv = jnp.where(swap, vals[jj], v)
                            ni = jnp.where(swap, eik, idxs[jj])
                            pi = jnp.where(swap, idxs[jj], eik)
                            nv_l.append(nv)
                            ni_l.append(ni)
                            v, eik = pv, pi
                    vals, idxs = tuple(nv_l), tuple(ni_l)
                return vals, idxs

            vals, idxs = lax.fori_loop(
                0, N_EXPERTS // UNROLL, expert_body, init
            )
            for k in range(NI):
                for j in range(TOPK):
                    w_v[j, sls[k]] = vals[k * TOPK + j]
                    i_v[j, sls[k]] = idxs[k * TOPK + j]
            return 0

        lax.fori_loop(0, NG // NI, group_body, 0)
        pltpu.sync_copy(w_v, wout_hbm.at[wid])
        pltpu.sync_copy(i_v, iout_hbm.at[wid])

    return sc_topk


SC_BLOCKS = 2


def kernel(x, weight):
    n_tokens = x.shape[0]
    sc_tok = SC_BLOCKS * BT
    tc_tok = n_tokens - sc_tok
    nb_tc = tc_tok // BT
    ch = sc_tok // NW
    probs2, wt, it = _tc_stage(x, weight, ch, nb_tc, SC_BLOCKS)
    scw, sci = _sc_topk_build(ch)(probs2)
    w1 = wt[:, :tc_tok].T
    i1 = it[:, :tc_tok].T
    w2 = scw.transpose(0, 2, 1).reshape(sc_tok, TOPK)
    i2 = sci.transpose(0, 2, 1).reshape(sc_tok, TOPK)
    return jnp.concatenate([w1, w2], 0), jnp.concatenate([i1, i2], 0)
